# baseline (speedup 1.0000x reference)
import dataclasses
import functools

import jax
import jax.numpy as jnp
from jax.experimental import pallas as pl
from jax.experimental.pallas import tpu as pltpu
from jax.experimental.pallas import tpu_sc as plsc

HIDDEN = 1024
FFN = 2048
NUM_EXPERTS = 16
TOP_K = 2
CHUNK = 1024
N_CHUNKS = FFN // CHUNK
ROWS_PER_BLOCK = 8


def _logits_kernel(x_ref, gw_ref, logits_ref):
    logits_ref[...] = jax.lax.dot_general(
        x_ref[...], gw_ref[...], dimension_numbers=(((1,), (1,)), ((), ())),
        preferred_element_type=jnp.float32)


def _combine_body(in_vmem, out_vmem):
    @pl.loop(0, ROWS_PER_BLOCK)
    def _(r):
        v = in_vmem[r]
        lane = jax.lax.iota(jnp.int32, 16)
        m1 = jnp.max(v)
        i1 = jnp.min(jnp.where(v >= m1, lane, NUM_EXPERTS))
        sel1 = lane == i1
        v2 = jnp.where(sel1, -3.0e38, v)
        m2 = jnp.max(v2)
        i2 = jnp.min(jnp.where(v2 >= m2, lane, NUM_EXPERTS))
        sel2 = lane == i2
        ev = jnp.exp(jnp.full((16,), m2 - m1, jnp.float32))
        w1 = 1.0 / (1.0 + ev)
        out_vmem[r] = jnp.where(sel1, w1, 0.0) + jnp.where(sel2, 1.0 - w1, 0.0)


def _sc_combine(logits):
    t = logits.shape[0]

    sc_params = pltpu.CompilerParams()
    if "needs_layout_passes" in pltpu.CompilerParams.__dataclass_fields__:
        sc_params = dataclasses.replace(sc_params, needs_layout_passes=False)

    @functools.partial(
        pl.kernel,
        out_type=jax.ShapeDtypeStruct((t, NUM_EXPERTS), jnp.float32),
        mesh=plsc.VectorSubcoreMesh(core_axis_name="c", subcore_axis_name="s"),
        compiler_params=sc_params,
    )
    def run(logits_hbm, out_hbm):
        pltpu.emit_pipeline(
            _combine_body,
            grid=(t // ROWS_PER_BLOCK,),
            in_specs=[pl.BlockSpec((ROWS_PER_BLOCK, NUM_EXPERTS),
                                   lambda i: (i, 0))],
            out_specs=[pl.BlockSpec((ROWS_PER_BLOCK, NUM_EXPERTS),
                                    lambda i: (i, 0))],
            core_axis_name=("c", "s"),
            dimension_semantics=(pltpu.PARALLEL,),
        )(logits_hbm, out_hbm)

    return run(logits)


def _moe_kernel(x_ref, combine_ref, wg_ref, wu_ref, wd_ref, out_ref):
    e = pl.program_id(0)
    c = pl.program_id(1)
    first = jnp.logical_and(e == 0, c == 0)

    x = x_ref[...].astype(jnp.bfloat16)
    gate = jnp.dot(x, wg_ref[0].astype(jnp.bfloat16),
                   preferred_element_type=jnp.float32)
    up = jnp.dot(x, wu_ref[0].astype(jnp.bfloat16),
                 preferred_element_type=jnp.float32)
    hidden = gate * jax.nn.sigmoid(gate) * up
    down = jnp.dot(hidden.astype(jnp.bfloat16), wd_ref[0].astype(jnp.bfloat16),
                   preferred_element_type=jnp.float32)
    combine = combine_ref[...]
    lane = jax.lax.broadcasted_iota(jnp.int32, combine.shape, 1)
    col = jnp.sum(jnp.where(lane == e, combine, 0.0), axis=-1, keepdims=True)
    contrib = col * down

    @pl.when(first)
    def _init():
        out_ref[...] = contrib

    @pl.when(jnp.logical_not(first))
    def _acc():
        out_ref[...] = out_ref[...] + contrib


@functools.partial(jax.jit, static_argnames=())
def kernel(hidden_states, gate_w, w_gate_up, w_down):
    b, s, d = hidden_states.shape
    t = b * s
    x = hidden_states.reshape(t, d)

    logits = pl.pallas_call(
        _logits_kernel,
        out_shape=jax.ShapeDtypeStruct((t, NUM_EXPERTS), jnp.float32),
    )(x, gate_w)

    combine = _sc_combine(logits)

    out = pl.pallas_call(
        _moe_kernel,
        grid=(NUM_EXPERTS, N_CHUNKS),
        in_specs=[
            pl.BlockSpec((t, d), lambda e, c: (0, 0)),
            pl.BlockSpec((t, NUM_EXPERTS), lambda e, c: (0, 0)),
            pl.BlockSpec((1, d, CHUNK), lambda e, c: (e, 0, c)),
            pl.BlockSpec((1, d, CHUNK), lambda e, c: (e, 0, N_CHUNKS + c)),
            pl.BlockSpec((1, CHUNK, d), lambda e, c: (e, c, 0)),
        ],
        out_specs=pl.BlockSpec((t, d), lambda e, c: (0, 0)),
        out_shape=jax.ShapeDtypeStruct((t, d), jnp.float32),
        compiler_params=pltpu.CompilerParams(
            dimension_semantics=("arbitrary", "arbitrary"),
        ),
    )(x, combine, w_gate_up, w_gate_up, w_down)

    return out.reshape(b, s, d), logits
